# Initial kernel scaffold; baseline (speedup 1.0000x reference)
import jax
import jax.numpy as jnp
from jax.experimental import pallas as pl


def kernel(shape_emb, col_emb, pos_emb, Wl1, Wr1, b1, g1, be1, Wl2, Wr2, b2, g2, be2, Wlin, blin, shape_id, colour_id, pos_id, edge_index, batch):
    raise NotImplementedError("write your pallas kernel here")


# speedup vs baseline: 1.4825x; 1.4825x over previous
import functools

import jax
import jax.numpy as jnp
from jax import lax
from jax.experimental import pallas as pl
from jax.experimental.pallas import tpu as pltpu
from jax.experimental.pallas import tpu_sc as plsc

N = 50000
E = 800000
EMB = 32
D = 96
HID = 96
B = 64
MAXPOS = 2048

NC = 2
NS = 16
NW = NC * NS

NP = 50176
NODE_BLK = 112
NODE_BLKS = 14
TPT = NP // NS

EP = 800768
EDGE_BLK = 128
EDGE_BLKS = (EP // NS) // EDGE_BLK
EPT = EP // NS

TBL = 2064
NCHUNK = 6

ROW_BLK = 400
GRID = N // ROW_BLK


def _row8(v, width):
  return jnp.zeros((8, width), jnp.float32).at[0].set(v)


def _k0_body(x_ref, w_ref, o_ref):
  o_ref[...] = jnp.dot(x_ref[...], w_ref[...],
                       preferred_element_type=jnp.float32)


def _table_matmul(x0, w1cat):
  return pl.pallas_call(
      _k0_body,
      out_shape=jax.ShapeDtypeStruct((TBL, 2 * HID), jnp.float32),
  )(x0, w1cat)


def _ka_body(tlr_hbm, sid_hbm, cid_hbm, pid_hbm, pl_hbm, r_hbm,
             idxs, g0, g1, g2, cb, rb, sem):
  c = lax.axis_index("c")
  s = lax.axis_index("s")
  wid = s * NC + c
  tile_base = wid * (NODE_BLKS * NODE_BLK)

  def block(bl, carry):
    base = tile_base + bl * NODE_BLK
    pltpu.sync_copy(sid_hbm.at[pl.ds(base, NODE_BLK)], idxs.at[0])
    pltpu.sync_copy(cid_hbm.at[pl.ds(base, NODE_BLK)], idxs.at[1])
    pltpu.sync_copy(pid_hbm.at[pl.ds(base, NODE_BLK)], idxs.at[2])
    d0 = pltpu.async_copy(tlr_hbm.at[idxs.at[0]], g0, sem)
    d0.wait()
    d1 = pltpu.async_copy(tlr_hbm.at[idxs.at[1]], g1, sem)
    d1.wait()
    d2 = pltpu.async_copy(tlr_hbm.at[idxs.at[2]], g2, sem)
    d2.wait()

    def row(r, carry2):
      for m in range(NCHUNK):
        v = (g0[r, pl.ds(16 * m, 16)] + g1[r, pl.ds(16 * m, 16)]
             + g2[r, pl.ds(16 * m, 16)])
        cb[m, r, :] = v
      for m in range(NCHUNK):
        v = (g0[r, pl.ds(96 + 16 * m, 16)] + g1[r, pl.ds(96 + 16 * m, 16)]
             + g2[r, pl.ds(96 + 16 * m, 16)])
        rb[r, pl.ds(16 * m, 16)] = v
      return carry2

    lax.fori_loop(0, NODE_BLK, row, 0)
    for m in range(NCHUNK):
      pltpu.sync_copy(cb.at[m], pl_hbm.at[m, pl.ds(base, NODE_BLK), :])
    pltpu.sync_copy(rb, r_hbm.at[pl.ds(base, NODE_BLK), :])
    return carry

  lax.fori_loop(0, NODE_BLKS, block, 0)


def _node_build(tlr, sidp, cidp, pidp):
  mesh = plsc.VectorSubcoreMesh(core_axis_name="c", subcore_axis_name="s",
                                num_cores=NC, num_subcores=NS)
  return pl.kernel(
      _ka_body,
      out_type=[
          jax.ShapeDtypeStruct((NCHUNK, NP, 16), jnp.float32),
          jax.ShapeDtypeStruct((NP, 2 * HID - D), jnp.float32),
      ],
      mesh=mesh,
      scratch_types=[
          pltpu.VMEM((3, NODE_BLK), jnp.int32),
          pltpu.VMEM((NODE_BLK, 2 * HID), jnp.float32),
          pltpu.VMEM((NODE_BLK, 2 * HID), jnp.float32),
          pltpu.VMEM((NODE_BLK, 2 * HID), jnp.float32),
          pltpu.VMEM((NCHUNK, NODE_BLK, 16), jnp.float32),
          pltpu.VMEM((NODE_BLK, HID), jnp.float32),
          pltpu.SemaphoreType.DMA,
      ],
      compiler_params=pltpu.CompilerParams(use_tc_tiling_on_sc=False),
  )(tlr, sidp, cidp, pidp)


ZBLK = 392


def _make_kb_body(stride, with_count):
  def body(*refs):
    if with_count:
      (tbl_hbm, src_hbm, dst_hbm, s_hbm, cnt_hbm,
       acc, idxs, rows, ones, zbuf, sem) = refs
    else:
      (tbl_hbm, src_hbm, dst_hbm, s_hbm,
       acc, idxs, rows, ones, zbuf, sem) = refs
    c = lax.axis_index("c")
    s = lax.axis_index("s")
    edge_base = s * EPT

    def zrow(r, carry):
      zbuf[r, :] = jnp.zeros((16,), jnp.float32)
      return carry
    lax.fori_loop(0, ZBLK, zrow, 0)
    if with_count:
      def orow(r, carry):
        ones[r, :] = jnp.ones((16,), jnp.float32)
        return carry
      lax.fori_loop(0, EDGE_BLK, orow, 0)

    def zero_acc():
      for z in range(TPT // ZBLK):
        pltpu.sync_copy(zbuf, acc.at[pl.ds(s * TPT + z * ZBLK, ZBLK)])

    def sweep(chunk):
      def block(bl, carry):
        off = edge_base + bl * EDGE_BLK
        pltpu.sync_copy(dst_hbm.at[pl.ds(off, EDGE_BLK)], idxs.at[1])
        if chunk is None:
          pltpu.sync_copy(ones, acc.at[idxs.at[1]], add=True)
        else:
          pltpu.sync_copy(src_hbm.at[pl.ds(off, EDGE_BLK)], idxs.at[0])

          def add_off(j, carry2):
            v = idxs[0, pl.ds(16 * j, 16)]
            idxs[0, pl.ds(16 * j, 16)] = v + chunk * stride
            return carry2
          lax.fori_loop(0, EDGE_BLK // 16, add_off, 0)
          d = pltpu.async_copy(tbl_hbm.at[idxs.at[0]], rows, sem)
          d.wait()
          pltpu.sync_copy(rows, acc.at[idxs.at[1]], add=True)
        return carry
      lax.fori_loop(0, EDGE_BLKS, block, 0)

    for k in range(3):
      chunk = 3 * c + k
      zero_acc()
      plsc.subcore_barrier()
      sweep(chunk)
      plsc.subcore_barrier()
      pltpu.sync_copy(acc.at[pl.ds(s * TPT, TPT)],
                      s_hbm.at[chunk, pl.ds(s * TPT, TPT), :])
      plsc.subcore_barrier()

    if with_count:
      @pl.when(c == 0)
      def _():
        zero_acc()
        plsc.subcore_barrier()
        sweep(None)
        plsc.subcore_barrier()
        pltpu.sync_copy(acc.at[pl.ds(s * TPT, TPT)],
                        cnt_hbm.at[pl.ds(s * TPT, TPT)])
  return body


def _edge_agg(table, srcp, dstp, stride, with_count):
  mesh = plsc.VectorSubcoreMesh(core_axis_name="c", subcore_axis_name="s",
                                num_cores=NC, num_subcores=NS)
  out_type = [jax.ShapeDtypeStruct((NCHUNK, NP, 16), jnp.float32)]
  if with_count:
    out_type.append(jax.ShapeDtypeStruct((NP, 16), jnp.float32))
  return pl.kernel(
      _make_kb_body(stride, with_count),
      out_type=out_type,
      mesh=mesh,
      scratch_types=[
          pltpu.VMEM_SHARED((NP, 16), jnp.float32),
          pltpu.VMEM((2, EDGE_BLK), jnp.int32),
          pltpu.VMEM((EDGE_BLK, 16), jnp.float32),
          pltpu.VMEM((EDGE_BLK, 16), jnp.float32),
          pltpu.VMEM((ZBLK, 16), jnp.float32),
          pltpu.SemaphoreType.DMA,
      ],
      compiler_params=pltpu.CompilerParams(use_tc_tiling_on_sc=False),
  )(table, srcp, dstp)


def _kc_body(s0, s1, s2, s3, s4, s5, r_ref, cnt_ref, b_ref,
             h_ref, sum_ref, sq_ref):
  i = pl.program_id(0)

  @pl.when(i == 0)
  def _():
    sum_ref[...] = jnp.zeros_like(sum_ref)
    sq_ref[...] = jnp.zeros_like(sq_ref)

  s_cat = jnp.concatenate(
      [s0[...], s1[...], s2[...], s3[...], s4[...], s5[...]], axis=1)
  cnt = jnp.maximum(cnt_ref[...][:, 0:1], 1.0)
  h = s_cat / cnt + r_ref[...] + b_ref[0:1, :]
  h_ref[...] = h
  z7 = jnp.zeros((7, HID), jnp.float32)
  sum_ref[...] += jnp.concatenate([jnp.sum(h, axis=0, keepdims=True), z7], 0)
  sq_ref[...] += jnp.concatenate([jnp.sum(h * h, axis=0, keepdims=True), z7],
                                 0)


def _dense_stats(s_chunks, r, cnt16, bpad):
  blk = lambda: pl.BlockSpec((ROW_BLK, 16), lambda i: (i, 0))
  return pl.pallas_call(
      _kc_body,
      grid=(GRID,),
      in_specs=[blk() for _ in range(6)] + [
          pl.BlockSpec((ROW_BLK, HID), lambda i: (i, 0)),
          pl.BlockSpec((ROW_BLK, 16), lambda i: (i, 0)),
          pl.BlockSpec((8, HID), lambda i: (0, 0)),
      ],
      out_specs=[
          pl.BlockSpec((ROW_BLK, HID), lambda i: (i, 0)),
          pl.BlockSpec((8, HID), lambda i: (0, 0)),
          pl.BlockSpec((8, HID), lambda i: (0, 0)),
      ],
      out_shape=[
          jax.ShapeDtypeStruct((N, HID), jnp.float32),
          jax.ShapeDtypeStruct((8, HID), jnp.float32),
          jax.ShapeDtypeStruct((8, HID), jnp.float32),
      ],
  )(*s_chunks, r, cnt16, bpad)


def _kd_body(h_ref, sc_ref, sh_ref, w_ref, q_ref, r2_ref):
  y = jnp.maximum(h_ref[...] * sc_ref[0:1, :] + sh_ref[0:1, :], 0.0)
  o = jnp.dot(y, w_ref[...], preferred_element_type=jnp.float32)
  for m in range(NCHUNK):
    q_ref[m, :, :] = o[:, 16 * m:16 * m + 16]
  r2_ref[...] = o[:, HID:]


def _norm_matmul(h1, scale, shift, w2cat):
  return pl.pallas_call(
      _kd_body,
      grid=(GRID,),
      in_specs=[
          pl.BlockSpec((ROW_BLK, HID), lambda i: (i, 0)),
          pl.BlockSpec((8, HID), lambda i: (0, 0)),
          pl.BlockSpec((8, HID), lambda i: (0, 0)),
          pl.BlockSpec((HID, 2 * HID), lambda i: (0, 0)),
      ],
      out_specs=[
          pl.BlockSpec((NCHUNK, ROW_BLK, 16), lambda i: (0, i, 0)),
          pl.BlockSpec((ROW_BLK, HID), lambda i: (i, 0)),
      ],
      out_shape=[
          jax.ShapeDtypeStruct((NCHUNK, N, 16), jnp.float32),
          jax.ShapeDtypeStruct((N, HID), jnp.float32),
      ],
  )(h1, scale, shift, w2cat)


def _kf_body(h_ref, sc_ref, sh_ref, bt_ref, wl_ref, bl_ref, o_ref, pc_ref):
  i = pl.program_id(0)

  @pl.when(i == 0)
  def _():
    pc_ref[...] = jnp.zeros_like(pc_ref)

  y = jnp.maximum(h_ref[...] * sc_ref[0:1, :] + sh_ref[0:1, :], 0.0)
  bt = bt_ref[0, 0, :]
  onehot = (bt[:, None] == lax.broadcasted_iota(jnp.int32, (ROW_BLK, B), 1)
            ).astype(jnp.float32)
  ycat = jnp.concatenate([y, jnp.ones((ROW_BLK, HID), jnp.float32)], axis=1)
  pc_ref[...] += lax.dot_general(onehot, ycat, (((0,), (0,)), ((), ())),
                                 preferred_element_type=jnp.float32)

  @pl.when(i == GRID - 1)
  def _():
    pc = pc_ref[...]
    pooled = pc[:, :HID] / jnp.maximum(pc[:, HID:], 1.0)
    o_ref[...] = jnp.dot(pooled, wl_ref[...],
                         preferred_element_type=jnp.float32) + bl_ref[0:1, :]


def _norm_pool_final(h2, scale, shift, batch3d, wlin_pad, blin_pad):
  return pl.pallas_call(
      _kf_body,
      grid=(GRID,),
      in_specs=[
          pl.BlockSpec((ROW_BLK, HID), lambda i: (i, 0)),
          pl.BlockSpec((8, HID), lambda i: (0, 0)),
          pl.BlockSpec((8, HID), lambda i: (0, 0)),
          pl.BlockSpec((1, 1, ROW_BLK), lambda i: (i, 0, 0)),
          pl.BlockSpec((HID, 128), lambda i: (0, 0)),
          pl.BlockSpec((8, 128), lambda i: (0, 0)),
      ],
      out_specs=pl.BlockSpec((B, 128), lambda i: (0, 0)),
      out_shape=jax.ShapeDtypeStruct((B, 128), jnp.float32),
      scratch_shapes=[pltpu.VMEM((B, 2 * HID), jnp.float32)],
  )(h2, scale, shift, batch3d, wlin_pad, blin_pad)


def kernel(shape_emb, col_emb, pos_emb, Wl1, Wr1, b1, g1, be1,
           Wl2, Wr2, b2, g2, be2, Wlin, blin,
           shape_id, colour_id, pos_id, edge_index, batch):
  f32 = jnp.float32

  sidp = jnp.pad(shape_id.astype(jnp.int32), (0, NP - N))
  cidp = jnp.pad(colour_id.astype(jnp.int32), (0, NP - N)) + 6
  pidp = jnp.pad(jnp.clip(pos_id, 0, MAXPOS).astype(jnp.int32),
                 (0, NP - N)) + 12

  ei = edge_index.astype(jnp.int32)
  srcp = jnp.concatenate([ei[0], jnp.zeros((EP - E,), jnp.int32)])
  dstp = jnp.concatenate([ei[1], jnp.full((EP - E,), N, jnp.int32)])

  x0 = jnp.zeros((TBL, D), f32)
  x0 = x0.at[0:6, 0:EMB].set(shape_emb)
  x0 = x0.at[6:12, EMB:2 * EMB].set(col_emb)
  x0 = x0.at[12:12 + MAXPOS + 1, 2 * EMB:].set(pos_emb)
  w1cat = jnp.concatenate([Wl1, Wr1], axis=1)

  tlr = _table_matmul(x0, w1cat)
  p_chunks, r_full = _node_build(tlr, sidp, cidp, pidp)
  r1 = r_full[:N, :]

  s1_out, cnt_out = _edge_agg(p_chunks.reshape(NCHUNK * NP, 16),
                              srcp, dstp, NP, True)
  cnt16 = cnt_out[:N, :]
  s1_chunks = [s1_out[m, :N, :] for m in range(NCHUNK)]

  h1, sm1, sq1 = _dense_stats(s1_chunks, r1, cnt16, _row8(b1, HID))
  mean1 = sm1[0] / N
  var1 = sq1[0] / N - mean1 * mean1
  scale1 = g1 / jnp.sqrt(var1 + 1e-5)
  shift1 = be1 - mean1 * scale1

  w2cat = jnp.concatenate([Wl2, Wr2], axis=1)
  q_chunks, r2 = _norm_matmul(h1, _row8(scale1, HID), _row8(shift1, HID),
                              w2cat)
  (s2_out,) = _edge_agg(
      jnp.pad(q_chunks, ((0, 0), (0, NP - N), (0, 0))).reshape(
          NCHUNK * NP, 16),
      srcp, dstp, NP, False)
  s2_chunks = [s2_out[m, :N, :] for m in range(NCHUNK)]

  h2, sm2, sq2 = _dense_stats(s2_chunks, r2, cnt16, _row8(b2, HID))
  mean2 = sm2[0] / N
  var2 = sq2[0] / N - mean2 * mean2
  scale2 = g2 / jnp.sqrt(var2 + 1e-5)
  shift2 = be2 - mean2 * scale2

  batch3d = batch.astype(jnp.int32).reshape(GRID, 1, ROW_BLK)
  wlin_pad = jnp.zeros((HID, 128), f32).at[:, 0:2].set(Wlin)
  blin_pad = jnp.zeros((8, 128), f32).at[0, 0:2].set(blin)
  out = _norm_pool_final(h2, _row8(scale2, HID), _row8(shift2, HID),
                         batch3d, wlin_pad, blin_pad)
  return out[:, 0:2]


# speedup vs baseline: 2.2375x; 1.5093x over previous
import functools

import jax
import jax.numpy as jnp
from jax import lax
from jax.experimental import pallas as pl
from jax.experimental.pallas import tpu as pltpu
from jax.experimental.pallas import tpu_sc as plsc

N = 50000
E = 800000
EMB = 32
D = 96
HID = 96
B = 64
MAXPOS = 2048

NC = 2
NS = 16
NW = NC * NS

NP = 50176
NODE_BLK = 112
NODE_BLKS = 14
TPT = NP // NS

EP = 802816
EDGE_BLK = 128
EDGE_BLKS = (EP // NW) // EDGE_BLK
EPT = EP // NW

TBL = 2064
NCHUNK = 3
CW = 32

ROW_BLK = 400
GRID = N // ROW_BLK


def _row8(v, width):
  return jnp.zeros((8, width), jnp.float32).at[0].set(v)


def _k0_body(x_ref, w_ref, o_ref):
  o_ref[...] = jnp.dot(x_ref[...], w_ref[...],
                       preferred_element_type=jnp.float32)


def _table_matmul(x0, w1cat):
  return pl.pallas_call(
      _k0_body,
      out_shape=jax.ShapeDtypeStruct((TBL, 2 * HID), jnp.float32),
  )(x0, w1cat)


def _ka_body(tlr_hbm, sid_hbm, cid_hbm, pid_hbm, pl_hbm, r_hbm,
             idxs, g0, g1, g2, cb, rb, sem):
  c = lax.axis_index("c")
  s = lax.axis_index("s")
  wid = s * NC + c
  tile_base = wid * (NODE_BLKS * NODE_BLK)

  def block(bl, carry):
    base = tile_base + bl * NODE_BLK
    pltpu.sync_copy(sid_hbm.at[pl.ds(base, NODE_BLK)], idxs.at[0])
    pltpu.sync_copy(cid_hbm.at[pl.ds(base, NODE_BLK)], idxs.at[1])
    pltpu.sync_copy(pid_hbm.at[pl.ds(base, NODE_BLK)], idxs.at[2])
    d0 = pltpu.async_copy(tlr_hbm.at[idxs.at[0]], g0, sem)
    d0.wait()
    d1 = pltpu.async_copy(tlr_hbm.at[idxs.at[1]], g1, sem)
    d1.wait()
    d2 = pltpu.async_copy(tlr_hbm.at[idxs.at[2]], g2, sem)
    d2.wait()

    def row(r, carry2):
      for m in range(6):
        v = (g0[r, pl.ds(16 * m, 16)] + g1[r, pl.ds(16 * m, 16)]
             + g2[r, pl.ds(16 * m, 16)])
        cb[m // 2, r, pl.ds(16 * (m % 2), 16)] = v
      for m in range(6):
        v = (g0[r, pl.ds(96 + 16 * m, 16)] + g1[r, pl.ds(96 + 16 * m, 16)]
             + g2[r, pl.ds(96 + 16 * m, 16)])
        rb[r, pl.ds(16 * m, 16)] = v
      return carry2

    lax.fori_loop(0, NODE_BLK, row, 0)
    for m in range(NCHUNK):
      pltpu.sync_copy(cb.at[m], pl_hbm.at[m, pl.ds(base, NODE_BLK), :])
    pltpu.sync_copy(rb, r_hbm.at[pl.ds(base, NODE_BLK), :])
    return carry

  lax.fori_loop(0, NODE_BLKS, block, 0)


def _node_build(tlr, sidp, cidp, pidp):
  mesh = plsc.VectorSubcoreMesh(core_axis_name="c", subcore_axis_name="s",
                                num_cores=NC, num_subcores=NS)
  return pl.kernel(
      _ka_body,
      out_type=[
          jax.ShapeDtypeStruct((NCHUNK, NP, CW), jnp.float32),
          jax.ShapeDtypeStruct((NP, 2 * HID - D), jnp.float32),
      ],
      mesh=mesh,
      scratch_types=[
          pltpu.VMEM((3, NODE_BLK), jnp.int32),
          pltpu.VMEM((NODE_BLK, 2 * HID), jnp.float32),
          pltpu.VMEM((NODE_BLK, 2 * HID), jnp.float32),
          pltpu.VMEM((NODE_BLK, 2 * HID), jnp.float32),
          pltpu.VMEM((NCHUNK, NODE_BLK, CW), jnp.float32),
          pltpu.VMEM((NODE_BLK, HID), jnp.float32),
          pltpu.SemaphoreType.DMA,
      ],
      compiler_params=pltpu.CompilerParams(use_tc_tiling_on_sc=False),
  )(tlr, sidp, cidp, pidp)


ZBLK = 392


def _make_kb_body(stride, with_count):
  def body(*refs):
    if with_count:
      (tbl_hbm, src_hbm, dst_hbm, s_hbm, cnt_hbm,
       acc, idxs, rows, ones, zbuf, sem) = refs
    else:
      (tbl_hbm, src_hbm, dst_hbm, s_hbm,
       acc, idxs, rows, ones, zbuf, sem) = refs
    c = lax.axis_index("c")
    s = lax.axis_index("s")
    edge_base = (c * NS + s) * EPT

    def zrow(r, carry):
      for t in range(CW // 16):
        zbuf[r, pl.ds(16 * t, 16)] = jnp.zeros((16,), jnp.float32)
      return carry
    lax.fori_loop(0, ZBLK, zrow, 0)
    if with_count:
      def orow(r, carry):
        for t in range(CW // 16):
          ones[r, pl.ds(16 * t, 16)] = jnp.ones((16,), jnp.float32)
        return carry
      lax.fori_loop(0, EDGE_BLK, orow, 0)

    def zero_acc():
      for z in range(TPT // ZBLK):
        pltpu.sync_copy(zbuf, acc.at[pl.ds(s * TPT + z * ZBLK, ZBLK)])

    def sweep(chunk):
      def block(bl, carry):
        off = edge_base + bl * EDGE_BLK
        pltpu.sync_copy(dst_hbm.at[pl.ds(off, EDGE_BLK)], idxs.at[1])
        if chunk is None:
          pltpu.sync_copy(ones, acc.at[idxs.at[1]], add=True)
        else:
          pltpu.sync_copy(src_hbm.at[pl.ds(off, EDGE_BLK)], idxs.at[0])

          def add_off(j, carry2):
            v = idxs[0, pl.ds(16 * j, 16)]
            idxs[0, pl.ds(16 * j, 16)] = v + chunk * stride
            return carry2
          lax.fori_loop(0, EDGE_BLK // 16, add_off, 0)
          d = pltpu.async_copy(tbl_hbm.at[idxs.at[0]], rows, sem)
          d.wait()
          pltpu.sync_copy(rows, acc.at[idxs.at[1]], add=True)
        return carry
      lax.fori_loop(0, EDGE_BLKS, block, 0)

    for chunk in range(NCHUNK):
      zero_acc()
      plsc.subcore_barrier()
      sweep(chunk)
      plsc.subcore_barrier()
      pltpu.sync_copy(acc.at[pl.ds(s * TPT, TPT)],
                      s_hbm.at[c, chunk, pl.ds(s * TPT, TPT), :])
      plsc.subcore_barrier()

    if with_count:
      zero_acc()
      plsc.subcore_barrier()
      sweep(None)
      plsc.subcore_barrier()
      pltpu.sync_copy(acc.at[pl.ds(s * TPT, TPT)],
                      cnt_hbm.at[c, pl.ds(s * TPT, TPT), :])
  return body


def _edge_agg(table, srcp, dstp, stride, with_count):
  mesh = plsc.VectorSubcoreMesh(core_axis_name="c", subcore_axis_name="s",
                                num_cores=NC, num_subcores=NS)
  out_type = [jax.ShapeDtypeStruct((NC, NCHUNK, NP, CW), jnp.float32)]
  if with_count:
    out_type.append(jax.ShapeDtypeStruct((NC, NP, CW), jnp.float32))
  return pl.kernel(
      _make_kb_body(stride, with_count),
      out_type=out_type,
      mesh=mesh,
      scratch_types=[
          pltpu.VMEM_SHARED((NP, CW), jnp.float32),
          pltpu.VMEM((2, EDGE_BLK), jnp.int32),
          pltpu.VMEM((EDGE_BLK, CW), jnp.float32),
          pltpu.VMEM((EDGE_BLK, CW), jnp.float32),
          pltpu.VMEM((ZBLK, CW), jnp.float32),
          pltpu.SemaphoreType.DMA,
      ],
      compiler_params=pltpu.CompilerParams(use_tc_tiling_on_sc=False),
  )(table, srcp, dstp)


def _kc_body(s0, s1, s2, s3, s4, s5, c0_ref, c1_ref, r_ref, b_ref,
             h_ref, sum_ref, sq_ref):
  i = pl.program_id(0)

  @pl.when(i == 0)
  def _():
    sum_ref[...] = jnp.zeros_like(sum_ref)
    sq_ref[...] = jnp.zeros_like(sq_ref)

  s_cat = jnp.concatenate(
      [s0[...] + s3[...], s1[...] + s4[...], s2[...] + s5[...]], axis=1)
  cnt = jnp.maximum(c0_ref[...][:, 0:1] + c1_ref[...][:, 0:1], 1.0)
  h = s_cat / cnt + r_ref[...] + b_ref[0:1, :]
  h_ref[...] = h
  z7 = jnp.zeros((7, HID), jnp.float32)
  sum_ref[...] += jnp.concatenate([jnp.sum(h, axis=0, keepdims=True), z7], 0)
  sq_ref[...] += jnp.concatenate([jnp.sum(h * h, axis=0, keepdims=True), z7],
                                 0)


def _dense_stats(s_chunks, cnt_halves, r, bpad):
  blk = lambda: pl.BlockSpec((ROW_BLK, CW), lambda i: (i, 0))
  return pl.pallas_call(
      _kc_body,
      grid=(GRID,),
      in_specs=[blk() for _ in range(8)] + [
          pl.BlockSpec((ROW_BLK, HID), lambda i: (i, 0)),
          pl.BlockSpec((8, HID), lambda i: (0, 0)),
      ],
      out_specs=[
          pl.BlockSpec((ROW_BLK, HID), lambda i: (i, 0)),
          pl.BlockSpec((8, HID), lambda i: (0, 0)),
          pl.BlockSpec((8, HID), lambda i: (0, 0)),
      ],
      out_shape=[
          jax.ShapeDtypeStruct((N, HID), jnp.float32),
          jax.ShapeDtypeStruct((8, HID), jnp.float32),
          jax.ShapeDtypeStruct((8, HID), jnp.float32),
      ],
  )(*s_chunks, *cnt_halves, r, bpad)


def _kd_body(h_ref, sc_ref, sh_ref, w_ref, q_ref, r2_ref):
  y = jnp.maximum(h_ref[...] * sc_ref[0:1, :] + sh_ref[0:1, :], 0.0)
  o = jnp.dot(y, w_ref[...], preferred_element_type=jnp.float32)
  for m in range(NCHUNK):
    q_ref[m, :, :] = o[:, CW * m:CW * m + CW]
  r2_ref[...] = o[:, HID:]


def _norm_matmul(h1, scale, shift, w2cat):
  return pl.pallas_call(
      _kd_body,
      grid=(GRID,),
      in_specs=[
          pl.BlockSpec((ROW_BLK, HID), lambda i: (i, 0)),
          pl.BlockSpec((8, HID), lambda i: (0, 0)),
          pl.BlockSpec((8, HID), lambda i: (0, 0)),
          pl.BlockSpec((HID, 2 * HID), lambda i: (0, 0)),
      ],
      out_specs=[
          pl.BlockSpec((NCHUNK, ROW_BLK, CW), lambda i: (0, i, 0)),
          pl.BlockSpec((ROW_BLK, HID), lambda i: (i, 0)),
      ],
      out_shape=[
          jax.ShapeDtypeStruct((NCHUNK, N, CW), jnp.float32),
          jax.ShapeDtypeStruct((N, HID), jnp.float32),
      ],
  )(h1, scale, shift, w2cat)


def _kf_body(h_ref, sc_ref, sh_ref, bt_ref, wl_ref, bl_ref, o_ref, pc_ref):
  i = pl.program_id(0)

  @pl.when(i == 0)
  def _():
    pc_ref[...] = jnp.zeros_like(pc_ref)

  y = jnp.maximum(h_ref[...] * sc_ref[0:1, :] + sh_ref[0:1, :], 0.0)
  bt = bt_ref[0, 0, :]
  onehot = (bt[:, None] == lax.broadcasted_iota(jnp.int32, (ROW_BLK, B), 1)
            ).astype(jnp.float32)
  ycat = jnp.concatenate([y, jnp.ones((ROW_BLK, HID), jnp.float32)], axis=1)
  pc_ref[...] += lax.dot_general(onehot, ycat, (((0,), (0,)), ((), ())),
                                 preferred_element_type=jnp.float32)

  @pl.when(i == GRID - 1)
  def _():
    pc = pc_ref[...]
    pooled = pc[:, :HID] / jnp.maximum(pc[:, HID:], 1.0)
    o_ref[...] = jnp.dot(pooled, wl_ref[...],
                         preferred_element_type=jnp.float32) + bl_ref[0:1, :]


def _norm_pool_final(h2, scale, shift, batch3d, wlin_pad, blin_pad):
  return pl.pallas_call(
      _kf_body,
      grid=(GRID,),
      in_specs=[
          pl.BlockSpec((ROW_BLK, HID), lambda i: (i, 0)),
          pl.BlockSpec((8, HID), lambda i: (0, 0)),
          pl.BlockSpec((8, HID), lambda i: (0, 0)),
          pl.BlockSpec((1, 1, ROW_BLK), lambda i: (i, 0, 0)),
          pl.BlockSpec((HID, 128), lambda i: (0, 0)),
          pl.BlockSpec((8, 128), lambda i: (0, 0)),
      ],
      out_specs=pl.BlockSpec((B, 128), lambda i: (0, 0)),
      out_shape=jax.ShapeDtypeStruct((B, 128), jnp.float32),
      scratch_shapes=[pltpu.VMEM((B, 2 * HID), jnp.float32)],
  )(h2, scale, shift, batch3d, wlin_pad, blin_pad)


def kernel(shape_emb, col_emb, pos_emb, Wl1, Wr1, b1, g1, be1,
           Wl2, Wr2, b2, g2, be2, Wlin, blin,
           shape_id, colour_id, pos_id, edge_index, batch):
  f32 = jnp.float32

  sidp = jnp.pad(shape_id.astype(jnp.int32), (0, NP - N))
  cidp = jnp.pad(colour_id.astype(jnp.int32), (0, NP - N)) + 6
  pidp = jnp.pad(jnp.clip(pos_id, 0, MAXPOS).astype(jnp.int32),
                 (0, NP - N)) + 12

  ei = edge_index.astype(jnp.int32)
  srcp = jnp.concatenate([ei[0], jnp.zeros((EP - E,), jnp.int32)])
  dstp = jnp.concatenate([ei[1], jnp.full((EP - E,), N, jnp.int32)])

  x0 = jnp.zeros((TBL, D), f32)
  x0 = x0.at[0:6, 0:EMB].set(shape_emb)
  x0 = x0.at[6:12, EMB:2 * EMB].set(col_emb)
  x0 = x0.at[12:12 + MAXPOS + 1, 2 * EMB:].set(pos_emb)
  w1cat = jnp.concatenate([Wl1, Wr1], axis=1)

  tlr = _table_matmul(x0, w1cat)
  p_chunks, r_full = _node_build(tlr, sidp, cidp, pidp)
  r1 = r_full[:N, :]

  s1_out, cnt_out = _edge_agg(p_chunks.reshape(NCHUNK * NP, CW),
                              srcp, dstp, NP, True)
  cnt_halves = [cnt_out[c, :N, :] for c in range(NC)]
  s1_chunks = [s1_out[c, m, :N, :] for c in range(NC) for m in range(NCHUNK)]

  h1, sm1, sq1 = _dense_stats(s1_chunks, cnt_halves, r1, _row8(b1, HID))
  mean1 = sm1[0] / N
  var1 = sq1[0] / N - mean1 * mean1
  scale1 = g1 / jnp.sqrt(var1 + 1e-5)
  shift1 = be1 - mean1 * scale1

  w2cat = jnp.concatenate([Wl2, Wr2], axis=1)
  q_chunks, r2 = _norm_matmul(h1, _row8(scale1, HID), _row8(shift1, HID),
                              w2cat)
  (s2_out,) = _edge_agg(
      jnp.pad(q_chunks, ((0, 0), (0, NP - N), (0, 0))).reshape(
          NCHUNK * NP, CW),
      srcp, dstp, NP, False)
  s2_chunks = [s2_out[c, m, :N, :] for c in range(NC) for m in range(NCHUNK)]

  h2, sm2, sq2 = _dense_stats(s2_chunks, cnt_halves, r2, _row8(b2, HID))
  mean2 = sm2[0] / N
  var2 = sq2[0] / N - mean2 * mean2
  scale2 = g2 / jnp.sqrt(var2 + 1e-5)
  shift2 = be2 - mean2 * scale2

  batch3d = batch.astype(jnp.int32).reshape(GRID, 1, ROW_BLK)
  wlin_pad = jnp.zeros((HID, 128), f32).at[:, 0:2].set(Wlin)
  blin_pad = jnp.zeros((8, 128), f32).at[0, 0:2].set(blin)
  out = _norm_pool_final(h2, _row8(scale2, HID), _row8(shift2, HID),
                         batch3d, wlin_pad, blin_pad)
  return out[:, 0:2]


# speedup vs baseline: 3.0540x; 1.3649x over previous
import functools

import jax
import jax.numpy as jnp
from jax import lax
from jax.experimental import pallas as pl
from jax.experimental.pallas import tpu as pltpu
from jax.experimental.pallas import tpu_sc as plsc

N = 50000
E = 800000
EMB = 32
D = 96
HID = 96
B = 64
MAXPOS = 2048

NC = 2
NS = 16
NW = NC * NS

NP = 50176
NODE_BLK = 112
NODE_BLKS = 14
TPT = NP // NS

EP = 802816
EDGE_BLK = 128
EDGE_BLKS = (EP // NW) // EDGE_BLK
EPT = EP // NW

TBL = 2064
NCHUNK = 3
CW = 32

ROW_BLK = 400
GRID = N // ROW_BLK


def _row8(v, width):
  return jnp.zeros((8, width), jnp.float32).at[0].set(v)


def _k0_body(x_ref, w_ref, o_ref):
  o_ref[...] = jnp.dot(x_ref[...], w_ref[...],
                       preferred_element_type=jnp.float32)


def _table_matmul(x0, w1cat):
  return pl.pallas_call(
      _k0_body,
      out_shape=jax.ShapeDtypeStruct((TBL, 2 * HID), jnp.float32),
  )(x0, w1cat)


def _ka_body(tlr_hbm, sid_hbm, cid_hbm, pid_hbm, pl_hbm, r_hbm,
             idxs, g0, g1, g2, cb, rb, sem):
  c = lax.axis_index("c")
  s = lax.axis_index("s")
  wid = s * NC + c
  tile_base = wid * (NODE_BLKS * NODE_BLK)

  def block(bl, carry):
    base = tile_base + bl * NODE_BLK
    pltpu.sync_copy(sid_hbm.at[pl.ds(base, NODE_BLK)], idxs.at[0])
    pltpu.sync_copy(cid_hbm.at[pl.ds(base, NODE_BLK)], idxs.at[1])
    pltpu.sync_copy(pid_hbm.at[pl.ds(base, NODE_BLK)], idxs.at[2])
    d0 = pltpu.async_copy(tlr_hbm.at[idxs.at[0]], g0, sem)
    d0.wait()
    d1 = pltpu.async_copy(tlr_hbm.at[idxs.at[1]], g1, sem)
    d1.wait()
    d2 = pltpu.async_copy(tlr_hbm.at[idxs.at[2]], g2, sem)
    d2.wait()

    def row(r, carry2):
      for m in range(6):
        v = (g0[r, pl.ds(16 * m, 16)] + g1[r, pl.ds(16 * m, 16)]
             + g2[r, pl.ds(16 * m, 16)])
        cb[m // 2, r, pl.ds(16 * (m % 2), 16)] = v
      for m in range(6):
        v = (g0[r, pl.ds(96 + 16 * m, 16)] + g1[r, pl.ds(96 + 16 * m, 16)]
             + g2[r, pl.ds(96 + 16 * m, 16)])
        rb[r, pl.ds(16 * m, 16)] = v
      return carry2

    lax.fori_loop(0, NODE_BLK, row, 0)
    for m in range(NCHUNK):
      pltpu.sync_copy(cb.at[m], pl_hbm.at[m, pl.ds(base, NODE_BLK), :])
    pltpu.sync_copy(rb, r_hbm.at[pl.ds(base, NODE_BLK), :])
    return carry

  lax.fori_loop(0, NODE_BLKS, block, 0)


def _node_build(tlr, sidp, cidp, pidp):
  mesh = plsc.VectorSubcoreMesh(core_axis_name="c", subcore_axis_name="s",
                                num_cores=NC, num_subcores=NS)
  return pl.kernel(
      _ka_body,
      out_type=[
          jax.ShapeDtypeStruct((NCHUNK, NP, CW), jnp.float32),
          jax.ShapeDtypeStruct((NP, 2 * HID - D), jnp.float32),
      ],
      mesh=mesh,
      scratch_types=[
          pltpu.VMEM((3, NODE_BLK), jnp.int32),
          pltpu.VMEM((NODE_BLK, 2 * HID), jnp.float32),
          pltpu.VMEM((NODE_BLK, 2 * HID), jnp.float32),
          pltpu.VMEM((NODE_BLK, 2 * HID), jnp.float32),
          pltpu.VMEM((NCHUNK, NODE_BLK, CW), jnp.float32),
          pltpu.VMEM((NODE_BLK, HID), jnp.float32),
          pltpu.SemaphoreType.DMA,
      ],
      compiler_params=pltpu.CompilerParams(use_tc_tiling_on_sc=False),
  )(tlr, sidp, cidp, pidp)


ZBLK = 196
NB2 = EDGE_BLKS // 2


def _make_kb_body(with_count):
  def body(*refs):
    if with_count:
      (tbl0, tbl1, tbl2, src_hbm, dst_hbm, s_hbm, cnt_hbm,
       acc, idxs, rows, ones, zbuf, si0, si1, sg0, sg1) = refs
    else:
      (tbl0, tbl1, tbl2, src_hbm, dst_hbm, s_hbm,
       acc, idxs, rows, ones, zbuf, si0, si1, sg0, sg1) = refs
    tbls = (tbl0, tbl1, tbl2)
    si = (si0, si1)
    sg = (sg0, sg1)
    c = lax.axis_index("c")
    s = lax.axis_index("s")
    edge_base = (c * NS + s) * EPT

    def zrow(r, carry):
      for t in range(CW // 16):
        zbuf[r, pl.ds(16 * t, 16)] = jnp.zeros((16,), jnp.float32)
      return carry
    lax.fori_loop(0, ZBLK, zrow, 0)
    if with_count:
      def orow(r, carry):
        for t in range(CW // 16):
          ones[r, pl.ds(16 * t, 16)] = jnp.ones((16,), jnp.float32)
        return carry
      lax.fori_loop(0, EDGE_BLK, orow, 0)

    def zero_acc():
      for z in range(TPT // ZBLK):
        pltpu.sync_copy(zbuf, acc.at[pl.ds(s * TPT + z * ZBLK, ZBLK)])

    def sweep(tbl):
      def idx_start(slot, off):
        if tbl is not None:
          pltpu.make_async_copy(src_hbm.at[pl.ds(off, EDGE_BLK)],
                                idxs.at[slot, 0], si[slot]).start()
        pltpu.make_async_copy(dst_hbm.at[pl.ds(off, EDGE_BLK)],
                              idxs.at[slot, 1], si[slot]).start()

      def idx_wait(slot, off):
        if tbl is not None:
          pltpu.make_async_copy(src_hbm.at[pl.ds(off, EDGE_BLK)],
                                idxs.at[slot, 0], si[slot]).wait()
        pltpu.make_async_copy(dst_hbm.at[pl.ds(off, EDGE_BLK)],
                              idxs.at[slot, 1], si[slot]).wait()

      idx_start(0, edge_base)
      idx_start(1, edge_base + EDGE_BLK)

      def pair(i, carry):
        off0 = edge_base + (2 * i) * EDGE_BLK
        gs = []
        for slot in range(2):
          idx_wait(slot, off0 + slot * EDGE_BLK)
          if tbl is not None:
            gs.append(pltpu.async_copy(tbl.at[idxs.at[slot, 0]],
                                       rows.at[slot], sg[slot]))

        for slot in range(2):
          if tbl is None:
            pltpu.sync_copy(ones, acc.at[idxs.at[slot, 1]], add=True)
          else:
            gs[slot].wait()
            pltpu.sync_copy(rows.at[slot], acc.at[idxs.at[slot, 1]],
                            add=True)

        @pl.when(i < NB2 - 1)
        def _():
          idx_start(0, off0 + 2 * EDGE_BLK)
          idx_start(1, off0 + 3 * EDGE_BLK)
        return carry
      lax.fori_loop(0, NB2, pair, 0)

    for chunk in range(NCHUNK):
      zero_acc()
      plsc.subcore_barrier()
      sweep(tbls[chunk])
      plsc.subcore_barrier()
      pltpu.sync_copy(acc.at[pl.ds(s * TPT, TPT)],
                      s_hbm.at[c, chunk, pl.ds(s * TPT, TPT), :])
      plsc.subcore_barrier()

    if with_count:
      zero_acc()
      plsc.subcore_barrier()
      sweep(None)
      plsc.subcore_barrier()
      pltpu.sync_copy(acc.at[pl.ds(s * TPT, TPT)],
                      cnt_hbm.at[c, pl.ds(s * TPT, TPT), :])
  return body


def _edge_agg(tables, srcp, dstp, with_count):
  mesh = plsc.VectorSubcoreMesh(core_axis_name="c", subcore_axis_name="s",
                                num_cores=NC, num_subcores=NS)
  out_type = [jax.ShapeDtypeStruct((NC, NCHUNK, NP, CW), jnp.float32)]
  if with_count:
    out_type.append(jax.ShapeDtypeStruct((NC, NP, CW), jnp.float32))
  return pl.kernel(
      _make_kb_body(with_count),
      out_type=out_type,
      mesh=mesh,
      scratch_types=[
          pltpu.VMEM_SHARED((NP, CW), jnp.float32),
          pltpu.VMEM((2, 2, EDGE_BLK), jnp.int32),
          pltpu.VMEM((2, EDGE_BLK, CW), jnp.float32),
          pltpu.VMEM((EDGE_BLK, CW), jnp.float32),
          pltpu.VMEM((ZBLK, CW), jnp.float32),
          pltpu.SemaphoreType.DMA,
          pltpu.SemaphoreType.DMA,
          pltpu.SemaphoreType.DMA,
          pltpu.SemaphoreType.DMA,
      ],
      compiler_params=pltpu.CompilerParams(use_tc_tiling_on_sc=False),
  )(tables[0], tables[1], tables[2], srcp, dstp)


def _kc_body(s0, s1, s2, s3, s4, s5, c0_ref, c1_ref, r_ref, b_ref,
             h_ref, sum_ref, sq_ref):
  i = pl.program_id(0)

  @pl.when(i == 0)
  def _():
    sum_ref[...] = jnp.zeros_like(sum_ref)
    sq_ref[...] = jnp.zeros_like(sq_ref)

  s_cat = jnp.concatenate(
      [s0[...] + s3[...], s1[...] + s4[...], s2[...] + s5[...]], axis=1)
  cnt = jnp.maximum(c0_ref[...][:, 0:1] + c1_ref[...][:, 0:1], 1.0)
  h = s_cat / cnt + r_ref[...] + b_ref[0:1, :]
  h_ref[...] = h
  z7 = jnp.zeros((7, HID), jnp.float32)
  sum_ref[...] += jnp.concatenate([jnp.sum(h, axis=0, keepdims=True), z7], 0)
  sq_ref[...] += jnp.concatenate([jnp.sum(h * h, axis=0, keepdims=True), z7],
                                 0)


def _dense_stats(s_chunks, cnt_halves, r, bpad):
  blk = lambda: pl.BlockSpec((ROW_BLK, CW), lambda i: (i, 0))
  return pl.pallas_call(
      _kc_body,
      grid=(GRID,),
      in_specs=[blk() for _ in range(8)] + [
          pl.BlockSpec((ROW_BLK, HID), lambda i: (i, 0)),
          pl.BlockSpec((8, HID), lambda i: (0, 0)),
      ],
      out_specs=[
          pl.BlockSpec((ROW_BLK, HID), lambda i: (i, 0)),
          pl.BlockSpec((8, HID), lambda i: (0, 0)),
          pl.BlockSpec((8, HID), lambda i: (0, 0)),
      ],
      out_shape=[
          jax.ShapeDtypeStruct((N, HID), jnp.float32),
          jax.ShapeDtypeStruct((8, HID), jnp.float32),
          jax.ShapeDtypeStruct((8, HID), jnp.float32),
      ],
  )(*s_chunks, *cnt_halves, r, bpad)


def _kd_body(h_ref, sc_ref, sh_ref, w_ref, q_ref, r2_ref):
  y = jnp.maximum(h_ref[...] * sc_ref[0:1, :] + sh_ref[0:1, :], 0.0)
  o = jnp.dot(y, w_ref[...], preferred_element_type=jnp.float32)
  for m in range(NCHUNK):
    q_ref[m, :, :] = o[:, CW * m:CW * m + CW]
  r2_ref[...] = o[:, HID:]


def _norm_matmul(h1, scale, shift, w2cat):
  return pl.pallas_call(
      _kd_body,
      grid=(GRID,),
      in_specs=[
          pl.BlockSpec((ROW_BLK, HID), lambda i: (i, 0)),
          pl.BlockSpec((8, HID), lambda i: (0, 0)),
          pl.BlockSpec((8, HID), lambda i: (0, 0)),
          pl.BlockSpec((HID, 2 * HID), lambda i: (0, 0)),
      ],
      out_specs=[
          pl.BlockSpec((NCHUNK, ROW_BLK, CW), lambda i: (0, i, 0)),
          pl.BlockSpec((ROW_BLK, HID), lambda i: (i, 0)),
      ],
      out_shape=[
          jax.ShapeDtypeStruct((NCHUNK, N, CW), jnp.float32),
          jax.ShapeDtypeStruct((N, HID), jnp.float32),
      ],
  )(h1, scale, shift, w2cat)


def _kf_body(h_ref, sc_ref, sh_ref, bt_ref, wl_ref, bl_ref, o_ref, pc_ref):
  i = pl.program_id(0)

  @pl.when(i == 0)
  def _():
    pc_ref[...] = jnp.zeros_like(pc_ref)

  y = jnp.maximum(h_ref[...] * sc_ref[0:1, :] + sh_ref[0:1, :], 0.0)
  bt = bt_ref[0, 0, :]
  onehot = (bt[:, None] == lax.broadcasted_iota(jnp.int32, (ROW_BLK, B), 1)
            ).astype(jnp.float32)
  ycat = jnp.concatenate([y, jnp.ones((ROW_BLK, HID), jnp.float32)], axis=1)
  pc_ref[...] += lax.dot_general(onehot, ycat, (((0,), (0,)), ((), ())),
                                 preferred_element_type=jnp.float32)

  @pl.when(i == GRID - 1)
  def _():
    pc = pc_ref[...]
    pooled = pc[:, :HID] / jnp.maximum(pc[:, HID:], 1.0)
    o_ref[...] = jnp.dot(pooled, wl_ref[...],
                         preferred_element_type=jnp.float32) + bl_ref[0:1, :]


def _norm_pool_final(h2, scale, shift, batch3d, wlin_pad, blin_pad):
  return pl.pallas_call(
      _kf_body,
      grid=(GRID,),
      in_specs=[
          pl.BlockSpec((ROW_BLK, HID), lambda i: (i, 0)),
          pl.BlockSpec((8, HID), lambda i: (0, 0)),
          pl.BlockSpec((8, HID), lambda i: (0, 0)),
          pl.BlockSpec((1, 1, ROW_BLK), lambda i: (i, 0, 0)),
          pl.BlockSpec((HID, 128), lambda i: (0, 0)),
          pl.BlockSpec((8, 128), lambda i: (0, 0)),
      ],
      out_specs=pl.BlockSpec((B, 128), lambda i: (0, 0)),
      out_shape=jax.ShapeDtypeStruct((B, 128), jnp.float32),
      scratch_shapes=[pltpu.VMEM((B, 2 * HID), jnp.float32)],
  )(h2, scale, shift, batch3d, wlin_pad, blin_pad)


def kernel(shape_emb, col_emb, pos_emb, Wl1, Wr1, b1, g1, be1,
           Wl2, Wr2, b2, g2, be2, Wlin, blin,
           shape_id, colour_id, pos_id, edge_index, batch):
  f32 = jnp.float32

  sidp = jnp.pad(shape_id.astype(jnp.int32), (0, NP - N))
  cidp = jnp.pad(colour_id.astype(jnp.int32), (0, NP - N)) + 6
  pidp = jnp.pad(jnp.clip(pos_id, 0, MAXPOS).astype(jnp.int32),
                 (0, NP - N)) + 12

  ei = edge_index.astype(jnp.int32)
  srcp = jnp.concatenate([ei[0], jnp.zeros((EP - E,), jnp.int32)])
  dstp = jnp.concatenate([ei[1], jnp.full((EP - E,), N, jnp.int32)])

  x0 = jnp.zeros((TBL, D), f32)
  x0 = x0.at[0:6, 0:EMB].set(shape_emb)
  x0 = x0.at[6:12, EMB:2 * EMB].set(col_emb)
  x0 = x0.at[12:12 + MAXPOS + 1, 2 * EMB:].set(pos_emb)
  w1cat = jnp.concatenate([Wl1, Wr1], axis=1)

  tlr = _table_matmul(x0, w1cat)
  p_chunks, r_full = _node_build(tlr, sidp, cidp, pidp)
  r1 = r_full[:N, :]

  s1_out, cnt_out = _edge_agg([p_chunks[m] for m in range(NCHUNK)],
                              srcp, dstp, True)
  cnt_halves = [cnt_out[c, :N, :] for c in range(NC)]
  s1_chunks = [s1_out[c, m, :N, :] for c in range(NC) for m in range(NCHUNK)]

  h1, sm1, sq1 = _dense_stats(s1_chunks, cnt_halves, r1, _row8(b1, HID))
  mean1 = sm1[0] / N
  var1 = sq1[0] / N - mean1 * mean1
  scale1 = g1 / jnp.sqrt(var1 + 1e-5)
  shift1 = be1 - mean1 * scale1

  w2cat = jnp.concatenate([Wl2, Wr2], axis=1)
  q_chunks, r2 = _norm_matmul(h1, _row8(scale1, HID), _row8(shift1, HID),
                              w2cat)
  qp = jnp.pad(q_chunks, ((0, 0), (0, NP - N), (0, 0)))
  (s2_out,) = _edge_agg([qp[m] for m in range(NCHUNK)], srcp, dstp, False)
  s2_chunks = [s2_out[c, m, :N, :] for c in range(NC) for m in range(NCHUNK)]

  h2, sm2, sq2 = _dense_stats(s2_chunks, cnt_halves, r2, _row8(b2, HID))
  mean2 = sm2[0] / N
  var2 = sq2[0] / N - mean2 * mean2
  scale2 = g2 / jnp.sqrt(var2 + 1e-5)
  shift2 = be2 - mean2 * scale2

  batch3d = batch.astype(jnp.int32).reshape(GRID, 1, ROW_BLK)
  wlin_pad = jnp.zeros((HID, 128), f32).at[:, 0:2].set(Wlin)
  blin_pad = jnp.zeros((8, 128), f32).at[0, 0:2].set(blin)
  out = _norm_pool_final(h2, _row8(scale2, HID), _row8(shift2, HID),
                         batch3d, wlin_pad, blin_pad)
  return out[:, 0:2]


# speedup vs baseline: 3.0862x; 1.0105x over previous
import functools

import jax
import jax.numpy as jnp
from jax import lax
from jax.experimental import pallas as pl
from jax.experimental.pallas import tpu as pltpu
from jax.experimental.pallas import tpu_sc as plsc

N = 50000
E = 800000
EMB = 32
D = 96
HID = 96
B = 64
MAXPOS = 2048

NC = 2
NS = 16
NW = NC * NS

NP = 50176
NODE_BLK = 112
NODE_BLKS = 14
TPT = NP // NS

EP = 802816
EDGE_BLK = 128
EDGE_BLKS = (EP // NW) // EDGE_BLK
EPT = EP // NW

TBL = 2064
NCHUNK = 3
CW = 32

ROW_BLK = 448
GRID = NP // ROW_BLK


def _row8(v, width):
  return jnp.zeros((8, width), jnp.float32).at[0].set(v)


def _k0_body(x_ref, w_ref, o_ref):
  o_ref[...] = jnp.dot(x_ref[...], w_ref[...],
                       preferred_element_type=jnp.float32)


def _table_matmul(x0, w1cat):
  return pl.pallas_call(
      _k0_body,
      out_shape=jax.ShapeDtypeStruct((TBL, 2 * HID), jnp.float32),
  )(x0, w1cat)


def _ka_body(tlr_hbm, sid_hbm, cid_hbm, pid_hbm, pl_hbm, r_hbm,
             idxs, g0, g1, g2, cb, rb, sem):
  c = lax.axis_index("c")
  s = lax.axis_index("s")
  wid = s * NC + c
  tile_base = wid * (NODE_BLKS * NODE_BLK)

  def block(bl, carry):
    base = tile_base + bl * NODE_BLK
    pltpu.sync_copy(sid_hbm.at[pl.ds(base, NODE_BLK)], idxs.at[0])
    pltpu.sync_copy(cid_hbm.at[pl.ds(base, NODE_BLK)], idxs.at[1])
    pltpu.sync_copy(pid_hbm.at[pl.ds(base, NODE_BLK)], idxs.at[2])
    d0 = pltpu.async_copy(tlr_hbm.at[idxs.at[0]], g0, sem)
    d0.wait()
    d1 = pltpu.async_copy(tlr_hbm.at[idxs.at[1]], g1, sem)
    d1.wait()
    d2 = pltpu.async_copy(tlr_hbm.at[idxs.at[2]], g2, sem)
    d2.wait()

    def row(r, carry2):
      for m in range(6):
        v = (g0[r, pl.ds(16 * m, 16)] + g1[r, pl.ds(16 * m, 16)]
             + g2[r, pl.ds(16 * m, 16)])
        cb[m // 2, r, pl.ds(16 * (m % 2), 16)] = v
      for m in range(6):
        v = (g0[r, pl.ds(96 + 16 * m, 16)] + g1[r, pl.ds(96 + 16 * m, 16)]
             + g2[r, pl.ds(96 + 16 * m, 16)])
        rb[r, pl.ds(16 * m, 16)] = v
      return carry2

    lax.fori_loop(0, NODE_BLK, row, 0)
    for m in range(NCHUNK):
      pltpu.sync_copy(cb.at[m], pl_hbm.at[m, pl.ds(base, NODE_BLK), :])
    pltpu.sync_copy(rb, r_hbm.at[pl.ds(base, NODE_BLK), :])
    return carry

  lax.fori_loop(0, NODE_BLKS, block, 0)


def _node_build(tlr, sidp, cidp, pidp):
  mesh = plsc.VectorSubcoreMesh(core_axis_name="c", subcore_axis_name="s",
                                num_cores=NC, num_subcores=NS)
  return pl.kernel(
      _ka_body,
      out_type=[
          jax.ShapeDtypeStruct((NCHUNK, NP, CW), jnp.float32),
          jax.ShapeDtypeStruct((NP, 2 * HID - D), jnp.float32),
      ],
      mesh=mesh,
      scratch_types=[
          pltpu.VMEM((3, NODE_BLK), jnp.int32),
          pltpu.VMEM((NODE_BLK, 2 * HID), jnp.float32),
          pltpu.VMEM((NODE_BLK, 2 * HID), jnp.float32),
          pltpu.VMEM((NODE_BLK, 2 * HID), jnp.float32),
          pltpu.VMEM((NCHUNK, NODE_BLK, CW), jnp.float32),
          pltpu.VMEM((NODE_BLK, HID), jnp.float32),
          pltpu.SemaphoreType.DMA,
      ],
      compiler_params=pltpu.CompilerParams(use_tc_tiling_on_sc=False),
  )(tlr, sidp, cidp, pidp)


ZBLK = 196
NB2 = EDGE_BLKS // 2


def _make_kb_body(with_count):
  def body(*refs):
    if with_count:
      (tbl0, tbl1, tbl2, src_hbm, dst_hbm, s_hbm, cnt_hbm,
       acc, idxs, rows, ones, zbuf, si0, si1, sg0, sg1) = refs
    else:
      (tbl0, tbl1, tbl2, src_hbm, dst_hbm, s_hbm,
       acc, idxs, rows, ones, zbuf, si0, si1, sg0, sg1) = refs
    tbls = (tbl0, tbl1, tbl2)
    si = (si0, si1)
    sg = (sg0, sg1)
    c = lax.axis_index("c")
    s = lax.axis_index("s")
    edge_base = (c * NS + s) * EPT

    def zrow(r, carry):
      for t in range(CW // 16):
        zbuf[r, pl.ds(16 * t, 16)] = jnp.zeros((16,), jnp.float32)
      return carry
    lax.fori_loop(0, ZBLK, zrow, 0)
    if with_count:
      def orow(r, carry):
        for t in range(CW // 16):
          ones[r, pl.ds(16 * t, 16)] = jnp.ones((16,), jnp.float32)
        return carry
      lax.fori_loop(0, EDGE_BLK, orow, 0)

    def zero_acc():
      for z in range(TPT // ZBLK):
        pltpu.sync_copy(zbuf, acc.at[pl.ds(s * TPT + z * ZBLK, ZBLK)])

    def sweep(tbl):
      def idx_start(slot, off):
        if tbl is not None:
          pltpu.make_async_copy(src_hbm.at[pl.ds(off, EDGE_BLK)],
                                idxs.at[slot, 0], si[slot]).start()
        pltpu.make_async_copy(dst_hbm.at[pl.ds(off, EDGE_BLK)],
                              idxs.at[slot, 1], si[slot]).start()

      def idx_wait(slot, off):
        if tbl is not None:
          pltpu.make_async_copy(src_hbm.at[pl.ds(off, EDGE_BLK)],
                                idxs.at[slot, 0], si[slot]).wait()
        pltpu.make_async_copy(dst_hbm.at[pl.ds(off, EDGE_BLK)],
                              idxs.at[slot, 1], si[slot]).wait()

      idx_start(0, edge_base)
      idx_start(1, edge_base + EDGE_BLK)

      def pair(i, carry):
        off0 = edge_base + (2 * i) * EDGE_BLK
        gs = []
        for slot in range(2):
          idx_wait(slot, off0 + slot * EDGE_BLK)
          if tbl is not None:
            gs.append(pltpu.async_copy(tbl.at[idxs.at[slot, 0]],
                                       rows.at[slot], sg[slot]))

        for slot in range(2):
          if tbl is None:
            pltpu.sync_copy(ones, acc.at[idxs.at[slot, 1]], add=True)
          else:
            gs[slot].wait()
            pltpu.sync_copy(rows.at[slot], acc.at[idxs.at[slot, 1]],
                            add=True)

        @pl.when(i < NB2 - 1)
        def _():
          idx_start(0, off0 + 2 * EDGE_BLK)
          idx_start(1, off0 + 3 * EDGE_BLK)
        return carry
      lax.fori_loop(0, NB2, pair, 0)

    for chunk in range(NCHUNK):
      zero_acc()
      plsc.subcore_barrier()
      sweep(tbls[chunk])
      plsc.subcore_barrier()
      pltpu.sync_copy(acc.at[pl.ds(s * TPT, TPT)],
                      s_hbm.at[c, chunk, pl.ds(s * TPT, TPT), :])
      plsc.subcore_barrier()

    if with_count:
      zero_acc()
      plsc.subcore_barrier()
      sweep(None)
      plsc.subcore_barrier()
      pltpu.sync_copy(acc.at[pl.ds(s * TPT, TPT)],
                      cnt_hbm.at[c, pl.ds(s * TPT, TPT), :])
  return body


def _edge_agg(tables, srcp, dstp, with_count):
  mesh = plsc.VectorSubcoreMesh(core_axis_name="c", subcore_axis_name="s",
                                num_cores=NC, num_subcores=NS)
  out_type = [jax.ShapeDtypeStruct((NC, NCHUNK, NP, CW), jnp.float32)]
  if with_count:
    out_type.append(jax.ShapeDtypeStruct((NC, NP, CW), jnp.float32))
  return pl.kernel(
      _make_kb_body(with_count),
      out_type=out_type,
      mesh=mesh,
      scratch_types=[
          pltpu.VMEM_SHARED((NP, CW), jnp.float32),
          pltpu.VMEM((2, 2, EDGE_BLK), jnp.int32),
          pltpu.VMEM((2, EDGE_BLK, CW), jnp.float32),
          pltpu.VMEM((EDGE_BLK, CW), jnp.float32),
          pltpu.VMEM((ZBLK, CW), jnp.float32),
          pltpu.SemaphoreType.DMA,
          pltpu.SemaphoreType.DMA,
          pltpu.SemaphoreType.DMA,
          pltpu.SemaphoreType.DMA,
      ],
      compiler_params=pltpu.CompilerParams(use_tc_tiling_on_sc=False),
  )(tables[0], tables[1], tables[2], srcp, dstp)


def _kc_body(s0, s1, s2, s3, s4, s5, c0_ref, c1_ref, r_ref, b_ref,
             h_ref, sum_ref, sq_ref):
  i = pl.program_id(0)

  @pl.when(i == 0)
  def _():
    sum_ref[...] = jnp.zeros_like(sum_ref)
    sq_ref[...] = jnp.zeros_like(sq_ref)

  s_cat = jnp.concatenate(
      [s0[...] + s3[...], s1[...] + s4[...], s2[...] + s5[...]], axis=1)
  cnt = jnp.maximum(c0_ref[...][:, 0:1] + c1_ref[...][:, 0:1], 1.0)
  rowid = lax.broadcasted_iota(jnp.int32, (ROW_BLK, 1), 0) + i * ROW_BLK
  maskf = (rowid < N).astype(jnp.float32)
  h = (s_cat / cnt + r_ref[...] + b_ref[0:1, :]) * maskf
  h_ref[...] = h
  z7 = jnp.zeros((7, HID), jnp.float32)
  sum_ref[...] += jnp.concatenate([jnp.sum(h, axis=0, keepdims=True), z7], 0)
  sq_ref[...] += jnp.concatenate([jnp.sum(h * h, axis=0, keepdims=True), z7],
                                 0)


def _dense_stats(s_chunks, cnt_halves, r, bpad):
  blk = lambda: pl.BlockSpec((ROW_BLK, CW), lambda i: (i, 0))
  return pl.pallas_call(
      _kc_body,
      grid=(GRID,),
      in_specs=[blk() for _ in range(8)] + [
          pl.BlockSpec((ROW_BLK, HID), lambda i: (i, 0)),
          pl.BlockSpec((8, HID), lambda i: (0, 0)),
      ],
      out_specs=[
          pl.BlockSpec((ROW_BLK, HID), lambda i: (i, 0)),
          pl.BlockSpec((8, HID), lambda i: (0, 0)),
          pl.BlockSpec((8, HID), lambda i: (0, 0)),
      ],
      out_shape=[
          jax.ShapeDtypeStruct((NP, HID), jnp.float32),
          jax.ShapeDtypeStruct((8, HID), jnp.float32),
          jax.ShapeDtypeStruct((8, HID), jnp.float32),
      ],
  )(*s_chunks, *cnt_halves, r, bpad)


def _kd_body(h_ref, sc_ref, sh_ref, w_ref, q_ref, r2_ref):
  y = jnp.maximum(h_ref[...] * sc_ref[0:1, :] + sh_ref[0:1, :], 0.0)
  o = jnp.dot(y, w_ref[...], preferred_element_type=jnp.float32)
  for m in range(NCHUNK):
    q_ref[m, :, :] = o[:, CW * m:CW * m + CW]
  r2_ref[...] = o[:, HID:]


def _norm_matmul(h1, scale, shift, w2cat):
  return pl.pallas_call(
      _kd_body,
      grid=(GRID,),
      in_specs=[
          pl.BlockSpec((ROW_BLK, HID), lambda i: (i, 0)),
          pl.BlockSpec((8, HID), lambda i: (0, 0)),
          pl.BlockSpec((8, HID), lambda i: (0, 0)),
          pl.BlockSpec((HID, 2 * HID), lambda i: (0, 0)),
      ],
      out_specs=[
          pl.BlockSpec((NCHUNK, ROW_BLK, CW), lambda i: (0, i, 0)),
          pl.BlockSpec((ROW_BLK, HID), lambda i: (i, 0)),
      ],
      out_shape=[
          jax.ShapeDtypeStruct((NCHUNK, NP, CW), jnp.float32),
          jax.ShapeDtypeStruct((NP, HID), jnp.float32),
      ],
  )(h1, scale, shift, w2cat)


def _kf_body(h_ref, sc_ref, sh_ref, bt_ref, wl_ref, bl_ref, o_ref, pc_ref):
  i = pl.program_id(0)

  @pl.when(i == 0)
  def _():
    pc_ref[...] = jnp.zeros_like(pc_ref)

  y = jnp.maximum(h_ref[...] * sc_ref[0:1, :] + sh_ref[0:1, :], 0.0)
  bt = bt_ref[0, 0, :]
  onehot = (bt[:, None] == lax.broadcasted_iota(jnp.int32, (ROW_BLK, B), 1)
            ).astype(jnp.float32)
  ycat = jnp.concatenate([y, jnp.ones((ROW_BLK, HID), jnp.float32)], axis=1)
  pc_ref[...] += lax.dot_general(onehot, ycat, (((0,), (0,)), ((), ())),
                                 preferred_element_type=jnp.float32)

  @pl.when(i == GRID - 1)
  def _():
    pc = pc_ref[...]
    pooled = pc[:, :HID] / jnp.maximum(pc[:, HID:], 1.0)
    o_ref[...] = jnp.dot(pooled, wl_ref[...],
                         preferred_element_type=jnp.float32) + bl_ref[0:1, :]


def _norm_pool_final(h2, scale, shift, batch3d, wlin_pad, blin_pad):
  return pl.pallas_call(
      _kf_body,
      grid=(GRID,),
      in_specs=[
          pl.BlockSpec((ROW_BLK, HID), lambda i: (i, 0)),
          pl.BlockSpec((8, HID), lambda i: (0, 0)),
          pl.BlockSpec((8, HID), lambda i: (0, 0)),
          pl.BlockSpec((1, 1, ROW_BLK), lambda i: (i, 0, 0)),
          pl.BlockSpec((HID, 128), lambda i: (0, 0)),
          pl.BlockSpec((8, 128), lambda i: (0, 0)),
      ],
      out_specs=pl.BlockSpec((B, 128), lambda i: (0, 0)),
      out_shape=jax.ShapeDtypeStruct((B, 128), jnp.float32),
      scratch_shapes=[pltpu.VMEM((B, 2 * HID), jnp.float32)],
  )(h2, scale, shift, batch3d, wlin_pad, blin_pad)


def kernel(shape_emb, col_emb, pos_emb, Wl1, Wr1, b1, g1, be1,
           Wl2, Wr2, b2, g2, be2, Wlin, blin,
           shape_id, colour_id, pos_id, edge_index, batch):
  f32 = jnp.float32

  sidp = jnp.pad(shape_id.astype(jnp.int32), (0, NP - N))
  cidp = jnp.pad(colour_id.astype(jnp.int32), (0, NP - N)) + 6
  pidp = jnp.pad(jnp.clip(pos_id, 0, MAXPOS).astype(jnp.int32),
                 (0, NP - N)) + 12

  ei = edge_index.astype(jnp.int32)
  srcp = jnp.concatenate([ei[0], jnp.zeros((EP - E,), jnp.int32)])
  dstp = jnp.concatenate([ei[1], jnp.full((EP - E,), N, jnp.int32)])

  x0 = jnp.zeros((TBL, D), f32)
  x0 = x0.at[0:6, 0:EMB].set(shape_emb)
  x0 = x0.at[6:12, EMB:2 * EMB].set(col_emb)
  x0 = x0.at[12:12 + MAXPOS + 1, 2 * EMB:].set(pos_emb)
  w1cat = jnp.concatenate([Wl1, Wr1], axis=1)

  tlr = _table_matmul(x0, w1cat)
  p_chunks, r1 = _node_build(tlr, sidp, cidp, pidp)

  s1_out, cnt_out = _edge_agg([p_chunks[m] for m in range(NCHUNK)],
                              srcp, dstp, True)
  cnt_halves = [cnt_out[c] for c in range(NC)]
  s1_chunks = [s1_out[c, m] for c in range(NC) for m in range(NCHUNK)]

  h1, sm1, sq1 = _dense_stats(s1_chunks, cnt_halves, r1, _row8(b1, HID))
  mean1 = sm1[0] / N
  var1 = sq1[0] / N - mean1 * mean1
  scale1 = g1 / jnp.sqrt(var1 + 1e-5)
  shift1 = be1 - mean1 * scale1

  w2cat = jnp.concatenate([Wl2, Wr2], axis=1)
  q_chunks, r2 = _norm_matmul(h1, _row8(scale1, HID), _row8(shift1, HID),
                              w2cat)
  (s2_out,) = _edge_agg([q_chunks[m] for m in range(NCHUNK)],
                        srcp, dstp, False)
  s2_chunks = [s2_out[c, m] for c in range(NC) for m in range(NCHUNK)]

  h2, sm2, sq2 = _dense_stats(s2_chunks, cnt_halves, r2, _row8(b2, HID))
  mean2 = sm2[0] / N
  var2 = sq2[0] / N - mean2 * mean2
  scale2 = g2 / jnp.sqrt(var2 + 1e-5)
  shift2 = be2 - mean2 * scale2

  batch3d = jnp.pad(batch.astype(jnp.int32), (0, NP - N),
                    constant_values=B).reshape(GRID, 1, ROW_BLK)
  wlin_pad = jnp.zeros((HID, 128), f32).at[:, 0:2].set(Wlin)
  blin_pad = jnp.zeros((8, 128), f32).at[0, 0:2].set(blin)
  out = _norm_pool_final(h2, _row8(scale2, HID), _row8(shift2, HID),
                         batch3d, wlin_pad, blin_pad)
  return out[:, 0:2]
